# Initial kernel scaffold; baseline (speedup 1.0000x reference)
import jax
import jax.numpy as jnp
from jax.experimental import pallas as pl


def kernel(query, key_, value, Wq, bq, Wk, bk, Wv, bv, Wo, bo):
    raise NotImplementedError("write your pallas kernel here")


# speedup vs baseline: 1.1934x; 1.1934x over previous
import math

import jax
import jax.numpy as jnp
from jax.experimental import pallas as pl

N, D, H = 2048, 2048, 16
DH = D // H
SCALE = 1.0 / math.sqrt(DH)
EPS = 1e-5

TM = 256
TQ = 256


def _proj_body(x_ref, w_ref, b_ref, o_ref):
    x = x_ref[...]
    w = w_ref[...]
    acc = jax.lax.dot_general(
        x, w, (((1,), (1,)), ((), ())), preferred_element_type=jnp.float32)
    o_ref[...] = acc + b_ref[...]


def _proj(x, W, b):
    return pl.pallas_call(
        _proj_body,
        grid=(N // TM,),
        in_specs=[
            pl.BlockSpec((TM, D), lambda i: (i, 0)),
            pl.BlockSpec((D, D), lambda i: (0, 0)),
            pl.BlockSpec((1, D), lambda i: (0, 0)),
        ],
        out_specs=pl.BlockSpec((TM, D), lambda i: (i, 0)),
        out_shape=jax.ShapeDtypeStruct((N, D), jnp.float32),
    )(x, W, b.reshape(1, D))


def _poincare(x):
    norm = jnp.sqrt(jnp.sum(x * x, axis=-1, keepdims=True))
    max_norm = 1.0 - EPS
    scale = jnp.where(norm > max_norm, max_norm / jnp.maximum(norm, 1e-12), 1.0)
    return x * scale


def _attn_body(q_ref, k_ref, v_ref, o_ref):
    q = _poincare(q_ref[...])
    k = _poincare(k_ref[...])
    s = jax.lax.dot_general(
        q, k, (((1,), (1,)), ((), ())), preferred_element_type=jnp.float32)
    s = s * SCALE
    m = jnp.max(s, axis=-1, keepdims=True)
    p = jnp.exp(s - m)
    l = jnp.sum(p, axis=-1, keepdims=True)
    o = jnp.dot(p, v_ref[...], preferred_element_type=jnp.float32)
    o_ref[...] = o / l


def _attention(q, k, v):
    return pl.pallas_call(
        _attn_body,
        grid=(H, N // TQ),
        in_specs=[
            pl.BlockSpec((TQ, DH), lambda h, i: (i, h)),
            pl.BlockSpec((N, DH), lambda h, i: (0, h)),
            pl.BlockSpec((N, DH), lambda h, i: (0, h)),
        ],
        out_specs=pl.BlockSpec((TQ, DH), lambda h, i: (i, h)),
        out_shape=jax.ShapeDtypeStruct((N, D), jnp.float32),
    )(q, k, v)


def kernel(query, key_, value, Wq, bq, Wk, bk, Wv, bv, Wo, bo):
    x_q = query.reshape(N, D)
    x_k = key_.reshape(N, D)
    x_v = value.reshape(N, D)
    q = _proj(x_q, Wq, bq)
    k = _proj(x_k, Wk, bk)
    v = _proj(x_v, Wv, bv)
    o = _attention(q, k, v)
    out = _proj(o, Wo, bo)
    return out.reshape(1, N, D)
